# baseline (speedup 1.0000x reference)
import jax
import jax.numpy as jnp
import numpy as np
from jax.experimental import pallas as pl
from jax.experimental.pallas import tpu as pltpu

D = 768; NB = 12; NH = 12; DH = 64; PS = 16; GP = 14; NP = 196; T = 197
E = 8; R = 128; BN = 192; FF = 3072; NC = 1000; MOE_START = 6; NBR = 6; BATCH = 4
TP = 200
ROWS = BATCH * TP
FQ = FF // 4
EH = E // 2
NS = 10

_VMEM_LIMIT = 100 * 1024 * 1024


def _dg(a, b, ca, cb):
    return jax.lax.dot_general(
        a, b, (((ca,), (cb,)), ((), ())), preferred_element_type=jnp.float32)


def _dgb(a, b, ca, cb):
    return jax.lax.dot_general(
        a.astype(jnp.bfloat16), b.astype(jnp.bfloat16),
        (((ca,), (cb,)), ((), ())), preferred_element_type=jnp.float32)


def _ln(x, g, b, eps=1e-6):
    m = x.mean(-1, keepdims=True)
    v = ((x - m) ** 2).mean(-1, keepdims=True)
    return (x - m) / jnp.sqrt(v + eps) * g + b


def _gelu(x):
    return 0.5 * x * (1.0 + jax.lax.erf(x * np.float32(1.0 / np.sqrt(2.0))))


def _embed_body(xp_ref, w_ref, b_ref, cp_ref, pos_ref, out_ref):
    xp = xp_ref[...].reshape(BATCH * NP, D)
    emb = _dgb(xp, w_ref[...], 1, 1) + b_ref[...]
    emb = emb.reshape(BATCH, NP, D) + pos_ref[...]
    cls = jnp.broadcast_to(cp_ref[...], (BATCH, 1, D))
    pad = jnp.zeros((BATCH, TP - 1 - NP, D), jnp.float32)
    out_ref[...] = jnp.concatenate([cls, emb, pad], axis=1)


def _attn_stage(hout, qkv_s, attn_s, h2_s, projw, projb, ln2g, ln2b):
    col = jax.lax.broadcasted_iota(jnp.int32, (TP, TP), 1)
    mask = jnp.where(col < T, 0.0, -1e30).astype(jnp.float32)
    for bb in range(BATCH):
        r0 = bb * TP
        for hp in range(NH // 2):
            c0 = hp * 2 * DH
            qp = qkv_s[r0:r0 + TP, c0:c0 + 2 * DH]
            kp = qkv_s[r0:r0 + TP, D + c0:D + c0 + 2 * DH]
            vp = qkv_s[r0:r0 + TP, 2 * D + c0:2 * D + c0 + 2 * DH]
            outs = []
            for hh in range(2):
                q = qp[:, hh * DH:(hh + 1) * DH]
                k = kp[:, hh * DH:(hh + 1) * DH]
                v = vp[:, hh * DH:(hh + 1) * DH]
                s = _dgb(q, k, 1, 1) + mask
                p = jax.nn.softmax(s, axis=-1)
                outs.append(_dgb(p, v, 1, 0))
            attn_s[r0:r0 + TP, c0:c0 + 2 * DH] = jnp.concatenate(outs, axis=1)
    h = hout[...].reshape(ROWS, D)
    h = h + _dgb(attn_s[...], projw[...].reshape(D, D), 1, 0) \
        + projb[...].reshape(1, D)
    hout[...] = h.reshape(BATCH, TP, D)
    h2_s[...] = _ln(h, ln2g[...].reshape(1, D), ln2b[...].reshape(1, D))


def _mlp_stage(s, hout, h2_s, fc1w, fc1b, fc2w, fc2b):
    hid = _gelu(_dgb(h2_s[...], fc1w[...].reshape(D, FQ), 1, 0)
                + fc1b[...].reshape(1, FQ))
    delta = _dgb(hid, fc2w[...].reshape(FQ, D), 1, 0)
    bias_on = jnp.where(s == 4, 1.0, 0.0).astype(jnp.float32)
    delta = delta + bias_on * fc2b[...].reshape(1, D)
    hout[...] = hout[...] + delta.reshape(BATCH, TP, D)


def _branch_half(h, half, qm, gamma, masks, bias, down, up, alpha):
    z = _dgb(h, qm, 1, 0)
    e = z * z
    e = e / (e.sum(-1, keepdims=True) + 1e-6)
    m = jax.nn.softmax(masks, axis=0)
    logits = _dgb(e * gamma, m, 1, 1) + bias
    probs = jax.nn.softmax(logits, axis=-1)
    hd = _gelu(_dgb(h, down.reshape(EH * BN, D), 1, 1))
    out = jnp.zeros((ROWS, D), jnp.float32)
    for ee in range(EH):
        y = _dgb(hd[:, ee * BN:(ee + 1) * BN], up[ee], 1, 1)
        out = out + probs[:, half * EH + ee:half * EH + ee + 1] * y
    row = jax.lax.broadcasted_iota(jnp.int32, (ROWS, 1), 0)
    tok = (row % TP) != 0
    return jnp.where(tok, alpha, 0.0) * out


def _blocks_body(ln1g, ln1b, qkvw, qkvb, projw, projb, ln2g, ln2b,
                 fc1w, fc1b, fc2w, fc2b, hin, qm, pm, gamma, masks, bias,
                 down, up, alpha, hout, aux, qkv_s, attn_s, h2_s, hn_s):
    i = pl.program_id(0)
    s = pl.program_id(1)
    j = jnp.clip(i - MOE_START, 0, NBR - 1)

    @pl.when(jnp.logical_and(i == 0, s == 0))
    def _():
        hout[...] = hin[...]
        aux[0] = 0.0

    @pl.when(s == 0)
    def _():
        h = hout[...].reshape(ROWS, D)
        hn = _ln(h, ln1g[...].reshape(1, D), ln1b[...].reshape(1, D))
        hn_s[...] = hn
        q = _dgb(hn, qkvw[...].reshape(D, D), 1, 0) + qkvb[...].reshape(1, D)
        qkv_s[:, 0:D] = q * np.float32(1.0 / np.sqrt(DH))

    @pl.when(s == 1)
    def _():
        qkv_s[:, D:2 * D] = _dgb(hn_s[...], qkvw[...].reshape(D, D), 1, 0) \
            + qkvb[...].reshape(1, D)

    @pl.when(s == 2)
    def _():
        qkv_s[:, 2 * D:3 * D] = _dgb(hn_s[...], qkvw[...].reshape(D, D), 1, 0) \
            + qkvb[...].reshape(1, D)

    @pl.when(s == 3)
    def _():
        _attn_stage(hout, qkv_s, attn_s, h2_s, projw, projb, ln2g, ln2b)

    @pl.when(jnp.logical_and(s >= 4, s <= 7))
    def _():
        _mlp_stage(s, hout, h2_s, fc1w, fc1b, fc2w, fc2b)

    moe = i >= MOE_START
    q2 = qm[...].reshape(D, R)
    p2 = pm[...].reshape(D, R)
    g2 = gamma[...].reshape(1, R)
    m2 = masks[...].reshape(E, R)
    b2 = bias[...].reshape(1, E)

    @pl.when(jnp.logical_and(moe, s == 8))
    def _():
        h = hout[...].reshape(ROWS, D)
        h2_s[...] = h
        upd = _branch_half(h, 0, q2, g2, m2, b2,
                           down[...].reshape(EH, BN, D),
                           up[...].reshape(EH, D, BN), alpha[j])
        hout[...] = (h + upd).reshape(BATCH, TP, D)
        eye = (jax.lax.broadcasted_iota(jnp.int32, (R, R), 0)
               == jax.lax.broadcasted_iota(jnp.int32, (R, R), 1)
               ).astype(jnp.float32)
        oq = _dg(q2, q2, 0, 0) - eye
        op = _dg(p2, p2, 0, 0) - eye
        aux[0] += 1e-3 * ((oq * oq).sum() + (op * op).sum())

    @pl.when(jnp.logical_and(moe, s == 9))
    def _():
        h = h2_s[...]
        upd = _branch_half(h, 1, q2, g2, m2, b2,
                           down[...].reshape(EH, BN, D),
                           up[...].reshape(EH, D, BN), alpha[j])
        hout[...] = hout[...] + upd.reshape(BATCH, TP, D)


def _head_body(h_ref, g_ref, b_ref, w_ref, hb_ref, out_ref):
    cls = h_ref[:, 0, :].reshape(BATCH, D)
    cls = _ln(cls, g_ref[...], b_ref[...])
    out_ref[...] = _dgb(cls, w_ref[...], 1, 0) + hb_ref[...]


def _qs(s):
    return jnp.clip(s, 0, 2)


def _ms(s):
    return jnp.clip(s - 4, 0, 3)


def _bs(s):
    return jnp.clip(s - 8, 0, 1)


def _bj(i):
    return jnp.clip(i - MOE_START, 0, NBR - 1)


def kernel(x, params):
    p = params
    bl = p['blocks']
    br = p['branches']

    xp = x.reshape(BATCH, 3, GP, PS, GP, PS).transpose(0, 2, 4, 1, 3, 5)
    xp = xp.reshape(BATCH, NP, 3 * PS * PS)
    cp = (p['cls'] + p['pos'][:, :1]).reshape(1, 1, D)
    pos_t = p['pos'][:, 1:]

    h0 = pl.pallas_call(
        _embed_body,
        out_shape=jax.ShapeDtypeStruct((BATCH, TP, D), jnp.float32),
        compiler_params=pltpu.CompilerParams(vmem_limit_bytes=_VMEM_LIMIT),
    )(xp, p['patch_w'], p['patch_b'].reshape(1, D), cp, pos_t)

    def r3(a):
        return a.reshape(a.shape[0], 1, a.shape[1])

    in_specs = [
        pl.BlockSpec((1, 1, D), lambda i, s: (i, 0, 0)),
        pl.BlockSpec((1, 1, D), lambda i, s: (i, 0, 0)),
        pl.BlockSpec((1, D, D), lambda i, s: (i, 0, _qs(s))),
        pl.BlockSpec((1, 1, D), lambda i, s: (i, 0, _qs(s))),
        pl.BlockSpec((1, D, D), lambda i, s: (i, 0, 0)),
        pl.BlockSpec((1, 1, D), lambda i, s: (i, 0, 0)),
        pl.BlockSpec((1, 1, D), lambda i, s: (i, 0, 0)),
        pl.BlockSpec((1, 1, D), lambda i, s: (i, 0, 0)),
        pl.BlockSpec((1, D, FQ), lambda i, s: (i, 0, _ms(s))),
        pl.BlockSpec((1, 1, FQ), lambda i, s: (i, 0, _ms(s))),
        pl.BlockSpec((1, FQ, D), lambda i, s: (i, _ms(s), 0)),
        pl.BlockSpec((1, 1, D), lambda i, s: (i, 0, 0)),
        pl.BlockSpec((BATCH, TP, D), lambda i, s: (0, 0, 0)),
        pl.BlockSpec((1, D, R), lambda i, s: (_bj(i), 0, 0)),
        pl.BlockSpec((1, D, R), lambda i, s: (_bj(i), 0, 0)),
        pl.BlockSpec((1, 1, R), lambda i, s: (_bj(i), 0, 0)),
        pl.BlockSpec((1, E, R), lambda i, s: (_bj(i), 0, 0)),
        pl.BlockSpec((1, 1, E), lambda i, s: (_bj(i), 0, 0)),
        pl.BlockSpec((1, EH, BN, D),
                     lambda i, s: (_bj(i), _bs(s), 0, 0)),
        pl.BlockSpec((1, EH, D, BN),
                     lambda i, s: (_bj(i), _bs(s), 0, 0)),
        pl.BlockSpec(memory_space=pltpu.SMEM),
    ]

    h2, aux = pl.pallas_call(
        _blocks_body,
        grid=(NB, NS),
        in_specs=in_specs,
        out_specs=[pl.BlockSpec((BATCH, TP, D), lambda i, s: (0, 0, 0)),
                   pl.BlockSpec(memory_space=pltpu.SMEM)],
        out_shape=[jax.ShapeDtypeStruct((BATCH, TP, D), jnp.float32),
                   jax.ShapeDtypeStruct((1,), jnp.float32)],
        scratch_shapes=[pltpu.VMEM((ROWS, 3 * D), jnp.float32),
                        pltpu.VMEM((ROWS, D), jnp.float32),
                        pltpu.VMEM((ROWS, D), jnp.float32),
                        pltpu.VMEM((ROWS, D), jnp.float32)],
        compiler_params=pltpu.CompilerParams(
            dimension_semantics=("arbitrary", "arbitrary"),
            vmem_limit_bytes=_VMEM_LIMIT),
    )(r3(bl['ln1_g']), r3(bl['ln1_b']), bl['qkv_w'], r3(bl['qkv_b']),
      bl['proj_w'], r3(bl['proj_b']), r3(bl['ln2_g']), r3(bl['ln2_b']),
      bl['fc1_w'], r3(bl['fc1_b']), bl['fc2_w'], r3(bl['fc2_b']),
      h0, br['Q'], br['P'], r3(br['gamma']), br['masks'],
      r3(br['bias']), br['down'], br['up'], br['alpha'])

    logits = pl.pallas_call(
        _head_body,
        out_shape=jax.ShapeDtypeStruct((BATCH, NC), jnp.float32),
        compiler_params=pltpu.CompilerParams(vmem_limit_bytes=_VMEM_LIMIT),
    )(h2, p['norm_g'].reshape(1, D), p['norm_b'].reshape(1, D),
      p['head_w'], p['head_b'].reshape(1, NC))

    return logits, aux.reshape(())


# speedup vs baseline: 1.1268x; 1.1268x over previous
import functools

import jax
import jax.numpy as jnp
import numpy as np
from jax.experimental import pallas as pl
from jax.experimental.pallas import tpu as pltpu

D = 768; NB = 12; NH = 12; DH = 64; PS = 16; GP = 14; NP = 196; T = 197
E = 8; R = 128; BN = 192; FF = 3072; NC = 1000; MOE_START = 6; NBR = 6; BATCH = 4
TP = 200
ROWS = BATCH * TP
FH = FF // 2
EH = E // 2

_VMEM_LIMIT = 100 * 1024 * 1024


def _dg(a, b, ca, cb):
    return jax.lax.dot_general(
        a, b, (((ca,), (cb,)), ((), ())), preferred_element_type=jnp.float32)


def _dgb(a, b, ca, cb):
    return jax.lax.dot_general(
        a.astype(jnp.bfloat16), b.astype(jnp.bfloat16),
        (((ca,), (cb,)), ((), ())), preferred_element_type=jnp.float32)


def _ln(x, g, b, eps=1e-6):
    m = x.mean(-1, keepdims=True)
    v = ((x - m) ** 2).mean(-1, keepdims=True)
    return (x - m) / jnp.sqrt(v + eps) * g + b


def _gelu(x):
    return 0.5 * x * (1.0 + jax.lax.erf(x * np.float32(1.0 / np.sqrt(2.0))))


def _embed_body(xp_ref, w_ref, b_ref, cp_ref, pos_ref, out_ref):
    xp = xp_ref[...].reshape(BATCH * NP, D)
    emb = _dgb(xp, w_ref[...], 1, 1) + b_ref[...]
    emb = emb.reshape(BATCH, NP, D) + pos_ref[...]
    cls = jnp.broadcast_to(cp_ref[...], (BATCH, 1, D))
    pad = jnp.zeros((BATCH, TP - 1 - NP, D), jnp.float32)
    out_ref[...] = jnp.concatenate([cls, emb, pad], axis=1)


def _attn_stage(hout, qkv_s, h2_s, ln1g, ln1b, qkvw, qkvb,
                projw, projb, ln2g, ln2b):
    h = hout[...].reshape(ROWS, D)
    hn = _ln(h, ln1g[...].reshape(1, D), ln1b[...].reshape(1, D))
    qkv = _dgb(hn, qkvw[...].reshape(D, 3 * D), 1, 0) \
        + qkvb[...].reshape(1, 3 * D)
    qs = np.float32(1.0 / np.sqrt(DH))
    qkv_s[:, 0:D] = (qkv[:, 0:D] * qs).astype(jnp.bfloat16)
    qkv_s[:, D:3 * D] = qkv[:, D:3 * D].astype(jnp.bfloat16)
    col = jax.lax.broadcasted_iota(jnp.int32, (TP, TP), 1)
    mask = jnp.where(col < T, 0.0, -1e30).astype(jnp.float32)
    pw = projw[...].reshape(D, D)
    for bb in range(BATCH):
        r0 = bb * TP
        acc = projb[...].reshape(1, D)
        for hp in range(NH // 2):
            c0 = hp * 2 * DH
            qp = qkv_s[r0:r0 + TP, c0:c0 + 2 * DH]
            kp = qkv_s[r0:r0 + TP, D + c0:D + c0 + 2 * DH]
            vp = qkv_s[r0:r0 + TP, 2 * D + c0:2 * D + c0 + 2 * DH]
            outs = []
            for hh in range(2):
                q = qp[:, hh * DH:(hh + 1) * DH]
                k = kp[:, hh * DH:(hh + 1) * DH]
                v = vp[:, hh * DH:(hh + 1) * DH]
                s = _dgb(q, k, 1, 1) + mask
                m = jnp.max(s, axis=-1, keepdims=True)
                p = jnp.exp(s - m)
                sm = p.sum(-1, keepdims=True)
                outs.append(_dgb(p, v, 1, 0) / sm)
            pair = jnp.concatenate(outs, axis=1)
            acc = acc + _dgb(pair, pw[c0:c0 + 2 * DH, :], 1, 0)
        hout[bb, :, :] = hout[bb, :, :] + acc
    h = hout[...].reshape(ROWS, D)
    h2_s[...] = _ln(h, ln2g[...].reshape(1, D),
                    ln2b[...].reshape(1, D)).astype(jnp.bfloat16)


def _mlp_stage(s, s0, hout, h2_s, fc1w, fc1b, fc2w, fc2b):
    hid = _gelu(_dgb(h2_s[...], fc1w[...].reshape(D, FH), 1, 0)
                + fc1b[...].reshape(1, FH))
    delta = _dgb(hid, fc2w[...].reshape(FH, D), 1, 0)
    bias_on = jnp.where(s == s0, 1.0, 0.0).astype(jnp.float32)
    delta = delta + bias_on * fc2b[...].reshape(1, D)
    hout[...] = hout[...] + delta.reshape(BATCH, TP, D)


def _branch_half(h, half, qm, gamma, masks, bias, down, up, alpha):
    z = _dgb(h, qm, 1, 0)
    e = z * z
    e = e / (e.sum(-1, keepdims=True) + 1e-6)
    m = jax.nn.softmax(masks, axis=0)
    logits = _dgb(e * gamma, m, 1, 1) + bias
    probs = jax.nn.softmax(logits, axis=-1)
    hd = _gelu(_dgb(h, down.reshape(EH * BN, D), 1, 1))
    out = jnp.zeros((ROWS, D), jnp.float32)
    for ee in range(EH):
        y = _dgb(hd[:, ee * BN:(ee + 1) * BN], up[ee], 1, 0)
        out = out + probs[:, half * EH + ee:half * EH + ee + 1] * y
    row = jax.lax.broadcasted_iota(jnp.int32, (ROWS, 1), 0)
    tok = (row % TP) != 0
    return jnp.where(tok, alpha, 0.0) * out


def _blocks_body(moe, *refs):
    if moe:
        (ln1g, ln1b, qkvw, qkvb, projw, projb, ln2g, ln2b,
         fc1w, fc1b, fc2w, fc2b, hin, qm, pm, gamma, masks, bias,
         down, up, alpha, hout, aux, qkv_s, h2_s) = refs
    else:
        (ln1g, ln1b, qkvw, qkvb, projw, projb, ln2g, ln2b,
         fc1w, fc1b, fc2w, fc2b, hin, hout, qkv_s, h2_s) = refs
    i = pl.program_id(0)
    s = pl.program_id(1)

    @pl.when(jnp.logical_and(i == 0, s == 0))
    def _():
        hout[...] = hin[...]
        if moe:
            aux[0] = 0.0

    @pl.when(s == 0)
    def _():
        _attn_stage(hout, qkv_s, h2_s, ln1g, ln1b, qkvw, qkvb,
                    projw, projb, ln2g, ln2b)

    @pl.when(jnp.logical_and(s >= 1, s <= 2))
    def _():
        _mlp_stage(s, 1, hout, h2_s, fc1w, fc1b, fc2w, fc2b)

    if moe:
        q2 = qm[...].reshape(D, R)
        p2 = pm[...].reshape(D, R)
        g2 = gamma[...].reshape(1, R)
        m2 = masks[...].reshape(E, R)
        b2 = bias[...].reshape(1, E)

        @pl.when(s == 3)
        def _():
            h = hout[...].reshape(ROWS, D)
            h2_s[...] = h.astype(jnp.bfloat16)
            upd = _branch_half(h, 0, q2, g2, m2, b2,
                               down[...].reshape(EH, BN, D),
                               up[...].reshape(EH, BN, D), alpha[i])
            hout[...] = (h + upd).reshape(BATCH, TP, D)
            eye = (jax.lax.broadcasted_iota(jnp.int32, (R, R), 0)
                   == jax.lax.broadcasted_iota(jnp.int32, (R, R), 1)
                   ).astype(jnp.float32)
            oq = _dg(q2, q2, 0, 0) - eye
            op = _dg(p2, p2, 0, 0) - eye
            aux[0] += 1e-3 * ((oq * oq).sum() + (op * op).sum())

        @pl.when(s == 4)
        def _():
            h = h2_s[...]
            upd = _branch_half(h, 1, q2, g2, m2, b2,
                               down[...].reshape(EH, BN, D),
                               up[...].reshape(EH, BN, D), alpha[i])
            hout[...] = hout[...] + upd.reshape(BATCH, TP, D)


def _head_body(h_ref, g_ref, b_ref, w_ref, hb_ref, out_ref):
    cls = h_ref[:, 0, :].reshape(BATCH, D)
    cls = _ln(cls, g_ref[...], b_ref[...])
    out_ref[...] = _dgb(cls, w_ref[...], 1, 0) + hb_ref[...]


def _block_w_specs(off):
    def ms(s):
        return jnp.clip(s - 1, 0, 1)
    return [
        pl.BlockSpec((1, 1, D), lambda i, s: (i + off, 0, 0)),
        pl.BlockSpec((1, 1, D), lambda i, s: (i + off, 0, 0)),
        pl.BlockSpec((1, D, 3 * D), lambda i, s: (i + off, 0, 0)),
        pl.BlockSpec((1, 1, 3 * D), lambda i, s: (i + off, 0, 0)),
        pl.BlockSpec((1, D, D), lambda i, s: (i + off, 0, 0)),
        pl.BlockSpec((1, 1, D), lambda i, s: (i + off, 0, 0)),
        pl.BlockSpec((1, 1, D), lambda i, s: (i + off, 0, 0)),
        pl.BlockSpec((1, 1, D), lambda i, s: (i + off, 0, 0)),
        pl.BlockSpec((1, D, FH), lambda i, s: (i + off, 0, ms(s))),
        pl.BlockSpec((1, 1, FH), lambda i, s: (i + off, 0, ms(s))),
        pl.BlockSpec((1, FH, D), lambda i, s: (i + off, ms(s), 0)),
        pl.BlockSpec((1, 1, D), lambda i, s: (i + off, 0, 0)),
    ]


def _h_spec():
    return pl.BlockSpec((BATCH, TP, D), lambda i, s: (0, 0, 0))


def _scratch():
    return [pltpu.VMEM((ROWS, 3 * D), jnp.bfloat16),
            pltpu.VMEM((ROWS, D), jnp.bfloat16)]


def kernel(x, params):
    p = params
    bl = p['blocks']
    br = p['branches']

    xp = x.reshape(BATCH, 3, GP, PS, GP, PS).transpose(0, 2, 4, 1, 3, 5)
    xp = xp.reshape(BATCH, NP, 3 * PS * PS)
    cp = (p['cls'] + p['pos'][:, :1]).reshape(1, 1, D)
    pos_t = p['pos'][:, 1:]

    h0 = pl.pallas_call(
        _embed_body,
        out_shape=jax.ShapeDtypeStruct((BATCH, TP, D), jnp.float32),
        compiler_params=pltpu.CompilerParams(vmem_limit_bytes=_VMEM_LIMIT),
    )(xp, p['patch_w'], p['patch_b'].reshape(1, D), cp, pos_t)

    def r3(a):
        return a.reshape(a.shape[0], 1, a.shape[1])

    block_args = [r3(bl['ln1_g']), r3(bl['ln1_b']), bl['qkv_w'],
                  r3(bl['qkv_b']), bl['proj_w'], r3(bl['proj_b']),
                  r3(bl['ln2_g']), r3(bl['ln2_b']), bl['fc1_w'],
                  r3(bl['fc1_b']), bl['fc2_w'], r3(bl['fc2_b'])]

    h1 = pl.pallas_call(
        functools.partial(_blocks_body, False),
        grid=(MOE_START, 3),
        in_specs=_block_w_specs(0) + [_h_spec()],
        out_specs=_h_spec(),
        out_shape=jax.ShapeDtypeStruct((BATCH, TP, D), jnp.float32),
        scratch_shapes=_scratch(),
        compiler_params=pltpu.CompilerParams(
            dimension_semantics=("arbitrary", "arbitrary"),
            vmem_limit_bytes=_VMEM_LIMIT),
    )(*block_args, h0)

    br_specs = [
        pl.BlockSpec((1, D, R), lambda i, s: (i, 0, 0)),
        pl.BlockSpec((1, D, R), lambda i, s: (i, 0, 0)),
        pl.BlockSpec((1, 1, R), lambda i, s: (i, 0, 0)),
        pl.BlockSpec((1, E, R), lambda i, s: (i, 0, 0)),
        pl.BlockSpec((1, 1, E), lambda i, s: (i, 0, 0)),
        pl.BlockSpec((1, EH, BN, D),
                     lambda i, s: (i, jnp.clip(s - 3, 0, 1), 0, 0)),
        pl.BlockSpec((1, EH, BN, D),
                     lambda i, s: (i, jnp.clip(s - 3, 0, 1), 0, 0)),
        pl.BlockSpec(memory_space=pltpu.SMEM),
    ]
    h2, aux = pl.pallas_call(
        functools.partial(_blocks_body, True),
        grid=(NBR, 5),
        in_specs=_block_w_specs(MOE_START) + [_h_spec()] + br_specs,
        out_specs=[_h_spec(), pl.BlockSpec(memory_space=pltpu.SMEM)],
        out_shape=[jax.ShapeDtypeStruct((BATCH, TP, D), jnp.float32),
                   jax.ShapeDtypeStruct((1,), jnp.float32)],
        scratch_shapes=_scratch(),
        compiler_params=pltpu.CompilerParams(
            dimension_semantics=("arbitrary", "arbitrary"),
            vmem_limit_bytes=_VMEM_LIMIT),
    )(*block_args, h1, br['Q'], br['P'], r3(br['gamma']), br['masks'],
      r3(br['bias']), br['down'], br['up'].transpose(0, 1, 3, 2), br['alpha'])

    logits = pl.pallas_call(
        _head_body,
        out_shape=jax.ShapeDtypeStruct((BATCH, NC), jnp.float32),
        compiler_params=pltpu.CompilerParams(vmem_limit_bytes=_VMEM_LIMIT),
    )(h2, p['norm_g'].reshape(1, D), p['norm_b'].reshape(1, D),
      p['head_w'], p['head_b'].reshape(1, NC))

    return logits, aux.reshape(())


# speedup vs baseline: 1.2049x; 1.0693x over previous
import functools

import jax
import jax.numpy as jnp
import numpy as np
from jax.experimental import pallas as pl
from jax.experimental.pallas import tpu as pltpu

D = 768; NB = 12; NH = 12; DH = 64; PS = 16; GP = 14; NP = 196; T = 197
E = 8; R = 128; BN = 192; FF = 3072; NC = 1000; MOE_START = 6; NBR = 6; BATCH = 4
TP = 200
ROWS = BATCH * TP
FQ = FF // 4
EH = E // 2

_VMEM_LIMIT = 100 * 1024 * 1024


def _dg(a, b, ca, cb):
    return jax.lax.dot_general(
        a, b, (((ca,), (cb,)), ((), ())), preferred_element_type=jnp.float32)


def _dgb(a, b, ca, cb):
    return jax.lax.dot_general(
        a.astype(jnp.bfloat16), b.astype(jnp.bfloat16),
        (((ca,), (cb,)), ((), ())), preferred_element_type=jnp.float32)


def _ln(x, g, b, eps=1e-6):
    m = x.mean(-1, keepdims=True)
    v = ((x - m) ** 2).mean(-1, keepdims=True)
    return (x - m) / jnp.sqrt(v + eps) * g + b


def _gelu(x):
    return 0.5 * x * (1.0 + jax.lax.erf(x * np.float32(1.0 / np.sqrt(2.0))))


def _embed_body(xp_ref, w_ref, b_ref, cp_ref, pos_ref, out_ref):
    xp = xp_ref[...].reshape(BATCH * NP, D)
    emb = _dgb(xp, w_ref[...], 1, 1) + b_ref[...]
    emb = emb.reshape(BATCH, NP, D) + pos_ref[...]
    cls = jnp.broadcast_to(cp_ref[...], (BATCH, 1, D))
    pad = jnp.zeros((BATCH, TP - 1 - NP, D), jnp.float32)
    out_ref[...] = jnp.concatenate([cls, emb, pad], axis=1)


def _attn_stage(hout, qkv_s, attn_s, h2_s, ln1g, ln1b, qkvw, qkvb,
                projw, projb, ln2g, ln2b):
    h = hout[...].reshape(ROWS, D)
    hn = _ln(h, ln1g[...].reshape(1, D), ln1b[...].reshape(1, D))
    qkv = _dgb(hn, qkvw[...].reshape(D, 3 * D), 1, 0) \
        + qkvb[...].reshape(1, 3 * D)
    qs = np.float32(1.0 / np.sqrt(DH))
    qkv_s[:, 0:D] = (qkv[:, 0:D] * qs).astype(jnp.bfloat16)
    qkv_s[:, D:3 * D] = qkv[:, D:3 * D].astype(jnp.bfloat16)
    col = jax.lax.broadcasted_iota(jnp.int32, (TP, TP), 1)
    mask = jnp.where(col < T, 0.0, -1e30).astype(jnp.float32)
    for bb in range(BATCH):
        r0 = bb * TP
        for hp in range(NH // 2):
            c0 = hp * 2 * DH
            qp = qkv_s[r0:r0 + TP, c0:c0 + 2 * DH]
            kp = qkv_s[r0:r0 + TP, D + c0:D + c0 + 2 * DH]
            vp = qkv_s[r0:r0 + TP, 2 * D + c0:2 * D + c0 + 2 * DH]
            outs = []
            for hh in range(2):
                q = qp[:, hh * DH:(hh + 1) * DH]
                k = kp[:, hh * DH:(hh + 1) * DH]
                v = vp[:, hh * DH:(hh + 1) * DH]
                s = _dgb(q, k, 1, 1) + mask
                m = jnp.max(s, axis=-1, keepdims=True)
                p = jnp.exp(s - m)
                sm = p.sum(-1, keepdims=True)
                outs.append((_dgb(p, v, 1, 0) / sm).astype(jnp.bfloat16))
            attn_s[r0:r0 + TP, c0:c0 + 2 * DH] = jnp.concatenate(outs, axis=1)
    h = h + _dgb(attn_s[...], projw[...].reshape(D, D), 1, 0) \
        + projb[...].reshape(1, D)
    hout[...] = h.reshape(BATCH, TP, D)
    h2_s[...] = _ln(h, ln2g[...].reshape(1, D),
                    ln2b[...].reshape(1, D)).astype(jnp.bfloat16)


def _mlp_stage(s, s0, hout, h2_s, fc1w, fc1b, fc2w, fc2b):
    hid = _gelu(_dgb(h2_s[...], fc1w[...].reshape(D, FQ), 1, 0)
                + fc1b[...].reshape(1, FQ))
    delta = _dgb(hid, fc2w[...].reshape(FQ, D), 1, 0)
    bias_on = jnp.where(s == s0, 1.0, 0.0).astype(jnp.float32)
    delta = delta + bias_on * fc2b[...].reshape(1, D)
    hout[...] = hout[...] + delta.reshape(BATCH, TP, D)


def _branch_half(h, half, qm, gamma, masks, bias, down, up, alpha):
    z = _dgb(h, qm, 1, 0)
    e = z * z
    e = e / (e.sum(-1, keepdims=True) + 1e-6)
    m = jax.nn.softmax(masks, axis=0)
    logits = _dgb(e * gamma, m, 1, 1) + bias
    probs = jax.nn.softmax(logits, axis=-1)
    hd = _gelu(_dgb(h, down.reshape(EH * BN, D), 1, 1))
    out = jnp.zeros((ROWS, D), jnp.float32)
    for ee in range(EH):
        y = _dgb(hd[:, ee * BN:(ee + 1) * BN], up[ee], 1, 0)
        out = out + probs[:, half * EH + ee:half * EH + ee + 1] * y
    row = jax.lax.broadcasted_iota(jnp.int32, (ROWS, 1), 0)
    tok = (row % TP) != 0
    return jnp.where(tok, alpha, 0.0) * out


def _blocks_body(moe, *refs):
    if moe:
        (ln1g, ln1b, qkvw, qkvb, projw, projb, ln2g, ln2b,
         fc1w, fc1b, fc2w, fc2b, hin, qm, pm, gamma, masks, bias,
         down, up, alpha, hout, aux, qkv_s, attn_s, h2_s) = refs
    else:
        (ln1g, ln1b, qkvw, qkvb, projw, projb, ln2g, ln2b,
         fc1w, fc1b, fc2w, fc2b, hin, hout, qkv_s, attn_s, h2_s) = refs
    i = pl.program_id(0)
    s = pl.program_id(1)

    @pl.when(jnp.logical_and(i == 0, s == 0))
    def _():
        hout[...] = hin[...]
        if moe:
            aux[0] = 0.0

    @pl.when(s == 0)
    def _():
        _attn_stage(hout, qkv_s, attn_s, h2_s, ln1g, ln1b, qkvw, qkvb,
                    projw, projb, ln2g, ln2b)

    @pl.when(jnp.logical_and(s >= 1, s <= 4))
    def _():
        _mlp_stage(s, 1, hout, h2_s, fc1w, fc1b, fc2w, fc2b)

    if moe:
        q2 = qm[...].reshape(D, R)
        p2 = pm[...].reshape(D, R)
        g2 = gamma[...].reshape(1, R)
        m2 = masks[...].reshape(E, R)
        b2 = bias[...].reshape(1, E)

        @pl.when(s == 5)
        def _():
            h = hout[...].reshape(ROWS, D)
            h2_s[...] = h.astype(jnp.bfloat16)
            upd = _branch_half(h, 0, q2, g2, m2, b2,
                               down[...].reshape(EH, BN, D),
                               up[...].reshape(EH, BN, D), alpha[i])
            hout[...] = (h + upd).reshape(BATCH, TP, D)
            eye = (jax.lax.broadcasted_iota(jnp.int32, (R, R), 0)
                   == jax.lax.broadcasted_iota(jnp.int32, (R, R), 1)
                   ).astype(jnp.float32)
            oq = _dg(q2, q2, 0, 0) - eye
            op = _dg(p2, p2, 0, 0) - eye
            aux[0] += 1e-3 * ((oq * oq).sum() + (op * op).sum())

        @pl.when(s == 6)
        def _():
            h = h2_s[...]
            upd = _branch_half(h, 1, q2, g2, m2, b2,
                               down[...].reshape(EH, BN, D),
                               up[...].reshape(EH, BN, D), alpha[i])
            hout[...] = hout[...] + upd.reshape(BATCH, TP, D)


def _head_body(h_ref, g_ref, b_ref, w_ref, hb_ref, out_ref):
    cls = h_ref[:, 0, :].reshape(BATCH, D)
    cls = _ln(cls, g_ref[...], b_ref[...])
    out_ref[...] = _dgb(cls, w_ref[...], 1, 0) + hb_ref[...]


def _block_w_specs(off):
    def ms(s):
        return jnp.clip(s - 1, 0, 3)
    return [
        pl.BlockSpec((1, 1, D), lambda i, s: (i + off, 0, 0)),
        pl.BlockSpec((1, 1, D), lambda i, s: (i + off, 0, 0)),
        pl.BlockSpec((1, D, 3 * D), lambda i, s: (i + off, 0, 0)),
        pl.BlockSpec((1, 1, 3 * D), lambda i, s: (i + off, 0, 0)),
        pl.BlockSpec((1, D, D), lambda i, s: (i + off, 0, 0)),
        pl.BlockSpec((1, 1, D), lambda i, s: (i + off, 0, 0)),
        pl.BlockSpec((1, 1, D), lambda i, s: (i + off, 0, 0)),
        pl.BlockSpec((1, 1, D), lambda i, s: (i + off, 0, 0)),
        pl.BlockSpec((1, D, FQ), lambda i, s: (i + off, 0, ms(s))),
        pl.BlockSpec((1, 1, FQ), lambda i, s: (i + off, 0, ms(s))),
        pl.BlockSpec((1, FQ, D), lambda i, s: (i + off, ms(s), 0)),
        pl.BlockSpec((1, 1, D), lambda i, s: (i + off, 0, 0)),
    ]


def _h_spec():
    return pl.BlockSpec((BATCH, TP, D), lambda i, s: (0, 0, 0))


def _scratch():
    return [pltpu.VMEM((ROWS, 3 * D), jnp.bfloat16),
            pltpu.VMEM((ROWS, D), jnp.bfloat16),
            pltpu.VMEM((ROWS, D), jnp.bfloat16)]


def kernel(x, params):
    p = params
    bl = p['blocks']
    br = p['branches']

    xp = x.reshape(BATCH, 3, GP, PS, GP, PS).transpose(0, 2, 4, 1, 3, 5)
    xp = xp.reshape(BATCH, NP, 3 * PS * PS)
    cp = (p['cls'] + p['pos'][:, :1]).reshape(1, 1, D)
    pos_t = p['pos'][:, 1:]

    h0 = pl.pallas_call(
        _embed_body,
        out_shape=jax.ShapeDtypeStruct((BATCH, TP, D), jnp.float32),
        compiler_params=pltpu.CompilerParams(vmem_limit_bytes=_VMEM_LIMIT),
    )(xp, p['patch_w'], p['patch_b'].reshape(1, D), cp, pos_t)

    def r3(a):
        return a.reshape(a.shape[0], 1, a.shape[1])

    block_args = [r3(bl['ln1_g']), r3(bl['ln1_b']), bl['qkv_w'],
                  r3(bl['qkv_b']), bl['proj_w'], r3(bl['proj_b']),
                  r3(bl['ln2_g']), r3(bl['ln2_b']), bl['fc1_w'],
                  r3(bl['fc1_b']), bl['fc2_w'], r3(bl['fc2_b'])]

    h1 = pl.pallas_call(
        functools.partial(_blocks_body, False),
        grid=(MOE_START, 5),
        in_specs=_block_w_specs(0) + [_h_spec()],
        out_specs=_h_spec(),
        out_shape=jax.ShapeDtypeStruct((BATCH, TP, D), jnp.float32),
        scratch_shapes=_scratch(),
        compiler_params=pltpu.CompilerParams(
            dimension_semantics=("arbitrary", "arbitrary"),
            vmem_limit_bytes=_VMEM_LIMIT),
    )(*block_args, h0)

    br_specs = [
        pl.BlockSpec((1, D, R), lambda i, s: (i, 0, 0)),
        pl.BlockSpec((1, D, R), lambda i, s: (i, 0, 0)),
        pl.BlockSpec((1, 1, R), lambda i, s: (i, 0, 0)),
        pl.BlockSpec((1, E, R), lambda i, s: (i, 0, 0)),
        pl.BlockSpec((1, 1, E), lambda i, s: (i, 0, 0)),
        pl.BlockSpec((1, EH, BN, D),
                     lambda i, s: (i, jnp.clip(s - 5, 0, 1), 0, 0)),
        pl.BlockSpec((1, EH, BN, D),
                     lambda i, s: (i, jnp.clip(s - 5, 0, 1), 0, 0)),
        pl.BlockSpec(memory_space=pltpu.SMEM),
    ]
    h2, aux = pl.pallas_call(
        functools.partial(_blocks_body, True),
        grid=(NBR, 7),
        in_specs=_block_w_specs(MOE_START) + [_h_spec()] + br_specs,
        out_specs=[_h_spec(), pl.BlockSpec(memory_space=pltpu.SMEM)],
        out_shape=[jax.ShapeDtypeStruct((BATCH, TP, D), jnp.float32),
                   jax.ShapeDtypeStruct((1,), jnp.float32)],
        scratch_shapes=_scratch(),
        compiler_params=pltpu.CompilerParams(
            dimension_semantics=("arbitrary", "arbitrary"),
            vmem_limit_bytes=_VMEM_LIMIT),
    )(*block_args, h1, br['Q'], br['P'], r3(br['gamma']), br['masks'],
      r3(br['bias']), br['down'], br['up'].transpose(0, 1, 3, 2), br['alpha'])

    logits = pl.pallas_call(
        _head_body,
        out_shape=jax.ShapeDtypeStruct((BATCH, NC), jnp.float32),
        compiler_params=pltpu.CompilerParams(vmem_limit_bytes=_VMEM_LIMIT),
    )(h2, p['norm_g'].reshape(1, D), p['norm_b'].reshape(1, D),
      p['head_w'], p['head_b'].reshape(1, NC))

    return logits, aux.reshape(())


# speedup vs baseline: 1.2206x; 1.0131x over previous
import functools

import jax
import jax.numpy as jnp
import numpy as np
from jax.experimental import pallas as pl
from jax.experimental.pallas import tpu as pltpu

D = 768; NB = 12; NH = 12; DH = 64; PS = 16; GP = 14; NP = 196; T = 197
E = 8; R = 128; BN = 192; FF = 3072; NC = 1000; MOE_START = 6; NBR = 6; BATCH = 4
TP = 200
ROWS = BATCH * TP
FQ = FF // 3
EH = E // 2

_VMEM_LIMIT = 100 * 1024 * 1024


def _dg(a, b, ca, cb):
    return jax.lax.dot_general(
        a, b, (((ca,), (cb,)), ((), ())), preferred_element_type=jnp.float32)


def _dgb(a, b, ca, cb):
    return jax.lax.dot_general(
        a.astype(jnp.bfloat16), b.astype(jnp.bfloat16),
        (((ca,), (cb,)), ((), ())), preferred_element_type=jnp.float32)


def _ln(x, g, b, eps=1e-6):
    m = x.mean(-1, keepdims=True)
    v = ((x - m) ** 2).mean(-1, keepdims=True)
    return (x - m) / jnp.sqrt(v + eps) * g + b


def _gelu(x):
    return 0.5 * x * (1.0 + jax.lax.erf(x * np.float32(1.0 / np.sqrt(2.0))))


def _embed_body(xp_ref, w_ref, b_ref, cp_ref, pos_ref, out_ref):
    xp = xp_ref[...].reshape(BATCH * NP, D)
    emb = _dgb(xp, w_ref[...], 1, 1) + b_ref[...]
    emb = emb.reshape(BATCH, NP, D) + pos_ref[...]
    cls = jnp.broadcast_to(cp_ref[...], (BATCH, 1, D))
    pad = jnp.zeros((BATCH, TP - 1 - NP, D), jnp.float32)
    out_ref[...] = jnp.concatenate([cls, emb, pad], axis=1)


def _attn_stage(hout, qkv_s, attn_s, h2_s, ln1g, ln1b, qkvw, qkvb,
                projw, projb, ln2g, ln2b):
    h = hout[...].reshape(ROWS, D)
    hn = _ln(h, ln1g[...].reshape(1, D), ln1b[...].reshape(1, D))
    qkv = _dgb(hn, qkvw[...].reshape(D, 3 * D), 1, 0) \
        + qkvb[...].reshape(1, 3 * D)
    qs = np.float32(1.0 / np.sqrt(DH))
    qkv_s[:, 0:D] = (qkv[:, 0:D] * qs).astype(jnp.bfloat16)
    qkv_s[:, D:3 * D] = qkv[:, D:3 * D].astype(jnp.bfloat16)
    col = jax.lax.broadcasted_iota(jnp.int32, (TP, TP), 1)
    mask = jnp.where(col < T, 0.0, -1e30).astype(jnp.float32)
    for bb in range(BATCH):
        r0 = bb * TP
        for hp in range(NH // 2):
            c0 = hp * 2 * DH
            qp = qkv_s[r0:r0 + TP, c0:c0 + 2 * DH]
            kp = qkv_s[r0:r0 + TP, D + c0:D + c0 + 2 * DH]
            vp = qkv_s[r0:r0 + TP, 2 * D + c0:2 * D + c0 + 2 * DH]
            outs = []
            for hh in range(2):
                q = qp[:, hh * DH:(hh + 1) * DH]
                k = kp[:, hh * DH:(hh + 1) * DH]
                v = vp[:, hh * DH:(hh + 1) * DH]
                s = _dgb(q, k, 1, 1) + mask
                m = jnp.max(s, axis=-1, keepdims=True)
                p = jnp.exp(s - m)
                sm = p.sum(-1, keepdims=True)
                outs.append((_dgb(p, v, 1, 0) / sm).astype(jnp.bfloat16))
            attn_s[r0:r0 + TP, c0:c0 + 2 * DH] = jnp.concatenate(outs, axis=1)
    h = h + _dgb(attn_s[...], projw[...].reshape(D, D), 1, 0) \
        + projb[...].reshape(1, D)
    hout[...] = h.reshape(BATCH, TP, D)
    h2_s[...] = _ln(h, ln2g[...].reshape(1, D),
                    ln2b[...].reshape(1, D)).astype(jnp.bfloat16)


def _mlp_stage(s, s0, hout, h2_s, fc1w, fc1b, fc2w, fc2b):
    hid = _gelu(_dgb(h2_s[...], fc1w[...].reshape(D, FQ), 1, 0)
                + fc1b[...].reshape(1, FQ))
    delta = _dgb(hid, fc2w[...].reshape(FQ, D), 1, 0)
    bias_on = jnp.where(s == s0, 1.0, 0.0).astype(jnp.float32)
    delta = delta + bias_on * fc2b[...].reshape(1, D)
    hout[...] = hout[...] + delta.reshape(BATCH, TP, D)


def _branch_half(h, half, qm, gamma, masks, bias, down, up, alpha):
    z = _dgb(h, qm, 1, 0)
    e = z * z
    e = e / (e.sum(-1, keepdims=True) + 1e-6)
    m = jax.nn.softmax(masks, axis=0)
    logits = _dgb(e * gamma, m, 1, 1) + bias
    probs = jax.nn.softmax(logits, axis=-1)
    hd = _gelu(_dgb(h, down.reshape(EH * BN, D), 1, 1))
    out = jnp.zeros((ROWS, D), jnp.float32)
    for ee in range(EH):
        y = _dgb(hd[:, ee * BN:(ee + 1) * BN], up[ee], 1, 0)
        out = out + probs[:, half * EH + ee:half * EH + ee + 1] * y
    row = jax.lax.broadcasted_iota(jnp.int32, (ROWS, 1), 0)
    tok = (row % TP) != 0
    return jnp.where(tok, alpha, 0.0) * out


def _blocks_body(moe, *refs):
    if moe:
        (ln1g, ln1b, qkvw, qkvb, projw, projb, ln2g, ln2b,
         fc1w, fc1b, fc2w, fc2b, hin, qm, pm, gamma, masks, bias,
         down, up, alpha, hout, aux, qkv_s, attn_s, h2_s) = refs
    else:
        (ln1g, ln1b, qkvw, qkvb, projw, projb, ln2g, ln2b,
         fc1w, fc1b, fc2w, fc2b, hin, hout, qkv_s, attn_s, h2_s) = refs
    i = pl.program_id(0)
    s = pl.program_id(1)

    @pl.when(jnp.logical_and(i == 0, s == 0))
    def _():
        hout[...] = hin[...]
        if moe:
            aux[0] = 0.0

    @pl.when(s == 0)
    def _():
        _attn_stage(hout, qkv_s, attn_s, h2_s, ln1g, ln1b, qkvw, qkvb,
                    projw, projb, ln2g, ln2b)

    @pl.when(jnp.logical_and(s >= 1, s <= 3))
    def _():
        _mlp_stage(s, 1, hout, h2_s, fc1w, fc1b, fc2w, fc2b)

    if moe:
        q2 = qm[...].reshape(D, R)
        p2 = pm[...].reshape(D, R)
        g2 = gamma[...].reshape(1, R)
        m2 = masks[...].reshape(E, R)
        b2 = bias[...].reshape(1, E)

        @pl.when(s == 4)
        def _():
            h = hout[...].reshape(ROWS, D)
            h2_s[...] = h.astype(jnp.bfloat16)
            upd = _branch_half(h, 0, q2, g2, m2, b2,
                               down[...].reshape(EH, BN, D),
                               up[...].reshape(EH, BN, D), alpha[i])
            hout[...] = (h + upd).reshape(BATCH, TP, D)
            eye = (jax.lax.broadcasted_iota(jnp.int32, (R, R), 0)
                   == jax.lax.broadcasted_iota(jnp.int32, (R, R), 1)
                   ).astype(jnp.float32)
            oq = _dg(q2, q2, 0, 0) - eye
            op = _dg(p2, p2, 0, 0) - eye
            aux[0] += 1e-3 * ((oq * oq).sum() + (op * op).sum())

        @pl.when(s == 5)
        def _():
            h = h2_s[...]
            upd = _branch_half(h, 1, q2, g2, m2, b2,
                               down[...].reshape(EH, BN, D),
                               up[...].reshape(EH, BN, D), alpha[i])
            hout[...] = hout[...] + upd.reshape(BATCH, TP, D)


def _head_body(h_ref, g_ref, b_ref, w_ref, hb_ref, out_ref):
    cls = h_ref[:, 0, :].reshape(BATCH, D)
    cls = _ln(cls, g_ref[...], b_ref[...])
    out_ref[...] = _dgb(cls, w_ref[...], 1, 0) + hb_ref[...]


def _block_w_specs(off):
    def ms(s):
        return jnp.clip(s - 1, 0, 2)
    return [
        pl.BlockSpec((1, 1, D), lambda i, s: (i + off, 0, 0)),
        pl.BlockSpec((1, 1, D), lambda i, s: (i + off, 0, 0)),
        pl.BlockSpec((1, D, 3 * D), lambda i, s: (i + off, 0, 0)),
        pl.BlockSpec((1, 1, 3 * D), lambda i, s: (i + off, 0, 0)),
        pl.BlockSpec((1, D, D), lambda i, s: (i + off, 0, 0)),
        pl.BlockSpec((1, 1, D), lambda i, s: (i + off, 0, 0)),
        pl.BlockSpec((1, 1, D), lambda i, s: (i + off, 0, 0)),
        pl.BlockSpec((1, 1, D), lambda i, s: (i + off, 0, 0)),
        pl.BlockSpec((1, D, FQ), lambda i, s: (i + off, 0, ms(s))),
        pl.BlockSpec((1, 1, FQ), lambda i, s: (i + off, 0, ms(s))),
        pl.BlockSpec((1, FQ, D), lambda i, s: (i + off, ms(s), 0)),
        pl.BlockSpec((1, 1, D), lambda i, s: (i + off, 0, 0)),
    ]


def _h_spec():
    return pl.BlockSpec((BATCH, TP, D), lambda i, s: (0, 0, 0))


def _scratch():
    return [pltpu.VMEM((ROWS, 3 * D), jnp.bfloat16),
            pltpu.VMEM((ROWS, D), jnp.bfloat16),
            pltpu.VMEM((ROWS, D), jnp.bfloat16)]


def kernel(x, params):
    p = params
    bl = p['blocks']
    br = p['branches']

    xp = x.reshape(BATCH, 3, GP, PS, GP, PS).transpose(0, 2, 4, 1, 3, 5)
    xp = xp.reshape(BATCH, NP, 3 * PS * PS)
    cp = (p['cls'] + p['pos'][:, :1]).reshape(1, 1, D)
    pos_t = p['pos'][:, 1:]

    h0 = pl.pallas_call(
        _embed_body,
        out_shape=jax.ShapeDtypeStruct((BATCH, TP, D), jnp.float32),
        compiler_params=pltpu.CompilerParams(vmem_limit_bytes=_VMEM_LIMIT),
    )(xp, p['patch_w'], p['patch_b'].reshape(1, D), cp, pos_t)

    def r3(a):
        return a.reshape(a.shape[0], 1, a.shape[1])

    block_args = [r3(bl['ln1_g']), r3(bl['ln1_b']), bl['qkv_w'],
                  r3(bl['qkv_b']), bl['proj_w'], r3(bl['proj_b']),
                  r3(bl['ln2_g']), r3(bl['ln2_b']), bl['fc1_w'],
                  r3(bl['fc1_b']), bl['fc2_w'], r3(bl['fc2_b'])]

    h1 = pl.pallas_call(
        functools.partial(_blocks_body, False),
        grid=(MOE_START, 4),
        in_specs=_block_w_specs(0) + [_h_spec()],
        out_specs=_h_spec(),
        out_shape=jax.ShapeDtypeStruct((BATCH, TP, D), jnp.float32),
        scratch_shapes=_scratch(),
        compiler_params=pltpu.CompilerParams(
            dimension_semantics=("arbitrary", "arbitrary"),
            vmem_limit_bytes=_VMEM_LIMIT),
    )(*block_args, h0)

    br_specs = [
        pl.BlockSpec((1, D, R), lambda i, s: (i, 0, 0)),
        pl.BlockSpec((1, D, R), lambda i, s: (i, 0, 0)),
        pl.BlockSpec((1, 1, R), lambda i, s: (i, 0, 0)),
        pl.BlockSpec((1, E, R), lambda i, s: (i, 0, 0)),
        pl.BlockSpec((1, 1, E), lambda i, s: (i, 0, 0)),
        pl.BlockSpec((1, EH, BN, D),
                     lambda i, s: (i, jnp.clip(s - 4, 0, 1), 0, 0)),
        pl.BlockSpec((1, EH, BN, D),
                     lambda i, s: (i, jnp.clip(s - 4, 0, 1), 0, 0)),
        pl.BlockSpec(memory_space=pltpu.SMEM),
    ]
    h2, aux = pl.pallas_call(
        functools.partial(_blocks_body, True),
        grid=(NBR, 6),
        in_specs=_block_w_specs(MOE_START) + [_h_spec()] + br_specs,
        out_specs=[_h_spec(), pl.BlockSpec(memory_space=pltpu.SMEM)],
        out_shape=[jax.ShapeDtypeStruct((BATCH, TP, D), jnp.float32),
                   jax.ShapeDtypeStruct((1,), jnp.float32)],
        scratch_shapes=_scratch(),
        compiler_params=pltpu.CompilerParams(
            dimension_semantics=("arbitrary", "arbitrary"),
            vmem_limit_bytes=_VMEM_LIMIT),
    )(*block_args, h1, br['Q'], br['P'], r3(br['gamma']), br['masks'],
      r3(br['bias']), br['down'], br['up'].transpose(0, 1, 3, 2), br['alpha'])

    logits = pl.pallas_call(
        _head_body,
        out_shape=jax.ShapeDtypeStruct((BATCH, NC), jnp.float32),
        compiler_params=pltpu.CompilerParams(vmem_limit_bytes=_VMEM_LIMIT),
    )(h2, p['norm_g'].reshape(1, D), p['norm_b'].reshape(1, D),
      p['head_w'], p['head_b'].reshape(1, NC))

    return logits, aux.reshape(())


# speedup vs baseline: 1.2812x; 1.0496x over previous
import functools

import jax
import jax.numpy as jnp
import numpy as np
from jax.experimental import pallas as pl
from jax.experimental.pallas import tpu as pltpu

D = 768; NB = 12; NH = 12; DH = 64; PS = 16; GP = 14; NP = 196; T = 197
E = 8; R = 128; BN = 192; FF = 3072; NC = 1000; MOE_START = 6; NBR = 6; BATCH = 4
TP = 200
ROWS = BATCH * TP
FQ = FF // 3
EH = E // 2

_VMEM_LIMIT = 100 * 1024 * 1024


def _dg(a, b, ca, cb):
    return jax.lax.dot_general(
        a, b, (((ca,), (cb,)), ((), ())), preferred_element_type=jnp.float32)


def _dgb(a, b, ca, cb):
    return jax.lax.dot_general(
        a.astype(jnp.bfloat16), b.astype(jnp.bfloat16),
        (((ca,), (cb,)), ((), ())), preferred_element_type=jnp.float32)


def _ln(x, g, b, eps=1e-6):
    m = x.mean(-1, keepdims=True)
    v = ((x - m) ** 2).mean(-1, keepdims=True)
    return (x - m) / jnp.sqrt(v + eps) * g + b


def _gelu(x):
    return 0.5 * x * (1.0 + jax.lax.erf(x * np.float32(1.0 / np.sqrt(2.0))))


def _embed_body(xp_ref, w_ref, b_ref, cp_ref, pos_ref, out_ref):
    xp = xp_ref[...].reshape(BATCH * NP, D)
    emb = _dgb(xp, w_ref[...], 1, 1) + b_ref[...]
    emb = emb.reshape(BATCH, NP, D) + pos_ref[...]
    cls = jnp.broadcast_to(cp_ref[...], (BATCH, 1, D))
    pad = jnp.zeros((BATCH, TP - 1 - NP, D), jnp.float32)
    out_ref[...] = jnp.concatenate([cls, emb, pad], axis=1)


def _attn_stage(hout, qkv_s, attn_s, h2_s, ln1g, ln1b, qkvw, qkvb,
                projw, projb, ln2g, ln2b):
    h = hout[...].reshape(ROWS, D)
    hn = _ln(h, ln1g[...].reshape(1, D), ln1b[...].reshape(1, D))
    qkv = _dgb(hn, qkvw[...].reshape(D, 3 * D), 1, 0) \
        + qkvb[...].reshape(1, 3 * D)
    qs = np.float32(1.0 / np.sqrt(DH))
    qkv_s[:, 0:D] = (qkv[:, 0:D] * qs).astype(jnp.bfloat16)
    qkv_s[:, D:3 * D] = qkv[:, D:3 * D].astype(jnp.bfloat16)
    col = jax.lax.broadcasted_iota(jnp.int32, (TP, TP), 1)
    mask = jnp.where(col < T, 0.0, -1e30).astype(jnp.float32)
    for bb in range(BATCH):
        r0 = bb * TP
        for hp in range(NH // 2):
            c0 = hp * 2 * DH
            qp = qkv_s[r0:r0 + TP, c0:c0 + 2 * DH]
            kp = qkv_s[r0:r0 + TP, D + c0:D + c0 + 2 * DH]
            vp = qkv_s[r0:r0 + TP, 2 * D + c0:2 * D + c0 + 2 * DH]
            outs = []
            for hh in range(2):
                q = qp[:, hh * DH:(hh + 1) * DH]
                k = kp[:, hh * DH:(hh + 1) * DH]
                v = vp[:, hh * DH:(hh + 1) * DH]
                s = _dgb(q, k, 1, 1) + mask
                m = jnp.max(s, axis=-1, keepdims=True)
                p = jnp.exp(s - m)
                sm = p.sum(-1, keepdims=True)
                outs.append((_dgb(p, v, 1, 0) / sm).astype(jnp.bfloat16))
            attn_s[r0:r0 + TP, c0:c0 + 2 * DH] = jnp.concatenate(outs, axis=1)
    h = h + _dgb(attn_s[...], projw[...].reshape(D, D), 1, 0) \
        + projb[...].reshape(1, D)
    hout[...] = h.reshape(BATCH, TP, D)
    h2_s[...] = _ln(h, ln2g[...].reshape(1, D),
                    ln2b[...].reshape(1, D)).astype(jnp.bfloat16)


def _mlp_stage(s, s0, hout, h2_s, fc1w, fc1b, fc2w, fc2b):
    hid = _gelu(_dgb(h2_s[...], fc1w[...].reshape(D, FQ), 1, 0)
                + fc1b[...].reshape(1, FQ))
    delta = _dgb(hid, fc2w[...].reshape(FQ, D), 1, 0)
    bias_on = jnp.where(s == s0, 1.0, 0.0).astype(jnp.float32)
    delta = delta + bias_on * fc2b[...].reshape(1, D)
    hout[...] = hout[...] + delta.reshape(BATCH, TP, D)


def _branch_half(h, half, qm, gamma, masks, bias, down, up, alpha):
    z = _dgb(h, qm, 1, 0)
    e = z * z
    e = e / (e.sum(-1, keepdims=True) + 1e-6)
    m = jax.nn.softmax(masks, axis=0)
    logits = _dgb(e * gamma, m, 1, 1) + bias
    probs = jax.nn.softmax(logits, axis=-1)
    hd = _gelu(_dgb(h, down.reshape(EH * BN, D), 1, 1))
    out = jnp.zeros((ROWS, D), jnp.float32)
    for ee in range(EH):
        y = _dgb(hd[:, ee * BN:(ee + 1) * BN], up[ee], 1, 0)
        out = out + probs[:, half * EH + ee:half * EH + ee + 1] * y
    row = jax.lax.broadcasted_iota(jnp.int32, (ROWS, 1), 0)
    tok = (row % TP) != 0
    return jnp.where(tok, alpha, 0.0) * out


def _blocks_body(moe, *refs):
    if moe:
        (ln1g, ln1b, qkvw, qkvb, projw, projb, ln2g, ln2b,
         fc1w, fc1b, fc2w, fc2b, hin, qm, pm, gamma, masks, bias,
         down, up, alpha, hout, aux, qkv_s, attn_s, h2_s) = refs
    else:
        (ln1g, ln1b, qkvw, qkvb, projw, projb, ln2g, ln2b,
         fc1w, fc1b, fc2w, fc2b, hin, hout, qkv_s, attn_s, h2_s) = refs
    i = pl.program_id(0)
    s = pl.program_id(1)

    @pl.when(jnp.logical_and(i == 0, s == 0))
    def _():
        hout[...] = hin[...]
        if moe:
            aux[0] = 0.0

    @pl.when(s == 0)
    def _():
        _attn_stage(hout, qkv_s, attn_s, h2_s, ln1g, ln1b, qkvw, qkvb,
                    projw, projb, ln2g, ln2b)

    @pl.when(jnp.logical_and(s >= 1, s <= 3))
    def _():
        _mlp_stage(s, 1, hout, h2_s, fc1w, fc1b, fc2w, fc2b)

    if moe:
        q2 = qm[...].reshape(D, R)
        p2 = pm[...].reshape(D, R)
        g2 = gamma[...].reshape(1, R)
        m2 = masks[...].reshape(E, R)
        b2 = bias[...].reshape(1, E)

        @pl.when(s == 4)
        def _():
            h = hout[...].reshape(ROWS, D)
            h2_s[...] = h.astype(jnp.bfloat16)
            upd = _branch_half(h, 0, q2, g2, m2, b2,
                               down[...].reshape(EH, BN, D),
                               up[...].reshape(EH, BN, D), alpha[i])
            hout[...] = (h + upd).reshape(BATCH, TP, D)
            eye = (jax.lax.broadcasted_iota(jnp.int32, (R, R), 0)
                   == jax.lax.broadcasted_iota(jnp.int32, (R, R), 1)
                   ).astype(jnp.float32)
            oq = _dg(q2, q2, 0, 0) - eye
            op = _dg(p2, p2, 0, 0) - eye
            aux[0] += 1e-3 * ((oq * oq).sum() + (op * op).sum())

        @pl.when(s == 5)
        def _():
            h = h2_s[...]
            upd = _branch_half(h, 1, q2, g2, m2, b2,
                               down[...].reshape(EH, BN, D),
                               up[...].reshape(EH, BN, D), alpha[i])
            hout[...] = hout[...] + upd.reshape(BATCH, TP, D)


def _head_body(h_ref, g_ref, b_ref, w_ref, hb_ref, out_ref):
    cls = h_ref[:, 0, :].reshape(BATCH, D)
    cls = _ln(cls, g_ref[...], b_ref[...])
    out_ref[...] = _dgb(cls, w_ref[...], 1, 0) + hb_ref[...]


def _block_w_specs(off, ni):
    def ms(s):
        return jnp.clip(s - 1, 0, 2)
    def ahead(i, s, s_sw):
        return jnp.minimum(i + jnp.where(s >= s_sw, 1, 0), ni - 1)
    return [
        pl.BlockSpec((1, 1, D), lambda i, s: (i + off, 0, 0)),
        pl.BlockSpec((1, 1, D), lambda i, s: (i + off, 0, 0)),
        pl.BlockSpec((1, D, 3 * D),
                     lambda i, s: (ahead(i, s, 1) + off, 0, 0)),
        pl.BlockSpec((1, 1, 3 * D), lambda i, s: (i + off, 0, 0)),
        pl.BlockSpec((1, D, D),
                     lambda i, s: (ahead(i, s, 2) + off, 0, 0)),
        pl.BlockSpec((1, 1, D), lambda i, s: (i + off, 0, 0)),
        pl.BlockSpec((1, 1, D), lambda i, s: (i + off, 0, 0)),
        pl.BlockSpec((1, 1, D), lambda i, s: (i + off, 0, 0)),
        pl.BlockSpec((1, D, FQ), lambda i, s: (i + off, 0, ms(s))),
        pl.BlockSpec((1, 1, FQ), lambda i, s: (i + off, 0, ms(s))),
        pl.BlockSpec((1, FQ, D), lambda i, s: (i + off, ms(s), 0)),
        pl.BlockSpec((1, 1, D), lambda i, s: (i + off, 0, 0)),
    ]


def _h_spec():
    return pl.BlockSpec((BATCH, TP, D), lambda i, s: (0, 0, 0))


def _scratch():
    return [pltpu.VMEM((ROWS, 3 * D), jnp.bfloat16),
            pltpu.VMEM((ROWS, D), jnp.bfloat16),
            pltpu.VMEM((ROWS, D), jnp.bfloat16)]


def kernel(x, params):
    p = params
    bl = p['blocks']
    br = p['branches']

    xp = x.reshape(BATCH, 3, GP, PS, GP, PS).transpose(0, 2, 4, 1, 3, 5)
    xp = xp.reshape(BATCH, NP, 3 * PS * PS)
    cp = (p['cls'] + p['pos'][:, :1]).reshape(1, 1, D)
    pos_t = p['pos'][:, 1:]

    h0 = pl.pallas_call(
        _embed_body,
        out_shape=jax.ShapeDtypeStruct((BATCH, TP, D), jnp.float32),
        compiler_params=pltpu.CompilerParams(vmem_limit_bytes=_VMEM_LIMIT),
    )(xp, p['patch_w'], p['patch_b'].reshape(1, D), cp, pos_t)

    def r3(a):
        return a.reshape(a.shape[0], 1, a.shape[1])

    block_args = [r3(bl['ln1_g']), r3(bl['ln1_b']), bl['qkv_w'],
                  r3(bl['qkv_b']), bl['proj_w'], r3(bl['proj_b']),
                  r3(bl['ln2_g']), r3(bl['ln2_b']), bl['fc1_w'],
                  r3(bl['fc1_b']), bl['fc2_w'], r3(bl['fc2_b'])]

    h1 = pl.pallas_call(
        functools.partial(_blocks_body, False),
        grid=(MOE_START, 4),
        in_specs=_block_w_specs(0, MOE_START) + [_h_spec()],
        out_specs=_h_spec(),
        out_shape=jax.ShapeDtypeStruct((BATCH, TP, D), jnp.float32),
        scratch_shapes=_scratch(),
        compiler_params=pltpu.CompilerParams(
            dimension_semantics=("arbitrary", "arbitrary"),
            vmem_limit_bytes=_VMEM_LIMIT),
    )(*block_args, h0)

    br_specs = [
        pl.BlockSpec((1, D, R), lambda i, s: (i, 0, 0)),
        pl.BlockSpec((1, D, R), lambda i, s: (i, 0, 0)),
        pl.BlockSpec((1, 1, R), lambda i, s: (i, 0, 0)),
        pl.BlockSpec((1, E, R), lambda i, s: (i, 0, 0)),
        pl.BlockSpec((1, 1, E), lambda i, s: (i, 0, 0)),
        pl.BlockSpec((1, EH, BN, D),
                     lambda i, s: (i, jnp.clip(s - 4, 0, 1), 0, 0)),
        pl.BlockSpec((1, EH, BN, D),
                     lambda i, s: (i, jnp.clip(s - 4, 0, 1), 0, 0)),
        pl.BlockSpec(memory_space=pltpu.SMEM),
    ]
    h2, aux = pl.pallas_call(
        functools.partial(_blocks_body, True),
        grid=(NBR, 6),
        in_specs=_block_w_specs(MOE_START, NBR) + [_h_spec()] + br_specs,
        out_specs=[_h_spec(), pl.BlockSpec(memory_space=pltpu.SMEM)],
        out_shape=[jax.ShapeDtypeStruct((BATCH, TP, D), jnp.float32),
                   jax.ShapeDtypeStruct((1,), jnp.float32)],
        scratch_shapes=_scratch(),
        compiler_params=pltpu.CompilerParams(
            dimension_semantics=("arbitrary", "arbitrary"),
            vmem_limit_bytes=_VMEM_LIMIT),
    )(*block_args, h1, br['Q'], br['P'], r3(br['gamma']), br['masks'],
      r3(br['bias']), br['down'], br['up'].transpose(0, 1, 3, 2), br['alpha'])

    logits = pl.pallas_call(
        _head_body,
        out_shape=jax.ShapeDtypeStruct((BATCH, NC), jnp.float32),
        compiler_params=pltpu.CompilerParams(vmem_limit_bytes=_VMEM_LIMIT),
    )(h2, p['norm_g'].reshape(1, D), p['norm_b'].reshape(1, D),
      p['head_w'], p['head_b'].reshape(1, NC))

    return logits, aux.reshape(())
